# baseline (speedup 1.0000x reference)
import jax
import jax.numpy as jnp
from jax import lax
from jax.experimental import pallas as pl
from jax.experimental.pallas import tpu as pltpu
from jax.experimental.pallas import tpu_sc as plsc

HIDDEN = 64
SEQ = 200
BATCH = 4096
EPS = 1e-12

NC = 2
NS = 16
NW = NC * NS
BPW = BATCH // NW
L = 16
NV = HIDDEN // L
PADW = 128

_GATHER_DNUMS = lax.GatherDimensionNumbers(
    offset_dims=(), collapsed_slice_dims=(0,), start_index_map=(0,))


def _lane_perm(x, perm):
    return lax.gather(x, perm[:, None], dimension_numbers=_GATHER_DNUMS,
                      slice_sizes=(1,),
                      mode=lax.GatherScatterMode.PROMISE_IN_BOUNDS)


def _allsum(x):
    lane = lax.iota(jnp.int32, L)
    for sh in (8, 4, 2, 1):
        x = x + _lane_perm(x, lane ^ sh)
    return x


def _rsqrt(v):
    i = lax.bitcast_convert_type(v, jnp.int32)
    y = lax.bitcast_convert_type(jnp.int32(0x5F3759DF) - (i >> 1),
                                 jnp.float32)
    for _ in range(2):
        y = y * (1.5 - 0.5 * v * y * y)
    return y


def _body(seqT_hbm, word_hbm, pos_hbm, gamma_hbm, beta_hbm, out_hbm,
          idx_v, rows_v, slab_v, pos_v, gam_v, bet_v, sg0, sg1, so0, so1):
    wid = lax.axis_index("s") * NC + lax.axis_index("c")
    b0 = wid * BPW
    sems_g = (sg0, sg1)
    sems_o = (so0, so1)

    pltpu.sync_copy(pos_hbm, pos_v)
    pltpu.sync_copy(gamma_hbm, gam_v)
    pltpu.sync_copy(beta_hbm, bet_v)
    gam = [gam_v[pl.ds(L * j, L)] for j in range(NV)]
    bet = [bet_v[pl.ds(L * j, L)] for j in range(NV)]
    lane = lax.iota(jnp.int32, L)
    idx_h = [L * j + lane for j in range(NV)]

    def stage_and_fire(s, pb):
        pltpu.sync_copy(seqT_hbm.at[s, pl.ds(b0, BPW)], idx_v.at[pb])
        pltpu.async_copy(word_hbm.at[idx_v.at[pb]], rows_v.at[pb],
                         sems_g[pb])

    def wait_gather(pb):
        pltpu.make_async_copy(word_hbm.at[idx_v.at[pb]], rows_v.at[pb],
                              sems_g[pb]).wait()

    def fire_out(s, pb):
        pltpu.async_copy(slab_v.at[pb],
                         out_hbm.at[s, slice(None), pl.ds(b0, BPW)],
                         sems_o[pb])

    def wait_out(pb):
        pltpu.make_async_copy(slab_v.at[pb],
                              out_hbm.at[0, slice(None), pl.ds(b0, BPW)],
                              sems_o[pb]).wait()

    def compute(s, pb):
        posv = [pos_v[s, pl.ds(L * j, L)] for j in range(NV)]

        @plsc.parallel_loop(0, BPW, 1, unroll=8)
        def _(t):
            x = [rows_v[pb, t, pl.ds(L * j, L)] + posv[j] for j in range(NV)]
            sm = (x[0] + x[1]) + (x[2] + x[3])
            q = (x[0] * x[0] + x[1] * x[1]) + (x[2] * x[2] + x[3] * x[3])
            mean = _allsum(sm) * (1.0 / HIDDEN)
            var = _allsum(q) * (1.0 / HIDDEN) - mean * mean
            rstd = _rsqrt(var + EPS)
            idx_b = jnp.full((L,), t, jnp.int32)
            for j in range(NV):
                xn = (x[j] - mean) * rstd
                plsc.store_scatter(slab_v.at[pb], [idx_h[j], idx_b],
                                   xn * gam[j] + bet[j])

    stage_and_fire(0, 0)

    def pos_step(s, pb):
        @pl.when(s < SEQ - 1)
        def _():
            stage_and_fire(s + 1, 1 - pb)

        @pl.when(s >= 2)
        def _():
            wait_out(pb)
        wait_gather(pb)
        compute(s, pb)
        fire_out(s, pb)

    def iter_m(m, _):
        pos_step(2 * m, 0)
        pos_step(2 * m + 1, 1)
        return 0

    lax.fori_loop(0, SEQ // 2, iter_m, 0)
    wait_out(0)
    wait_out(1)


def kernel(seq, word_table, pos_table, gamma, beta):
    seqT = seq.astype(jnp.int32).T
    word128 = jnp.pad(word_table, ((0, 0), (0, PADW - HIDDEN)))
    mesh = plsc.VectorSubcoreMesh(core_axis_name="c", subcore_axis_name="s",
                                  num_cores=NC, num_subcores=NS)
    k = pl.kernel(
        _body,
        out_type=jax.ShapeDtypeStruct((SEQ, HIDDEN, BATCH), jnp.float32),
        mesh=mesh,
        scratch_types=[
            pltpu.VMEM((2, BPW), jnp.int32),
            pltpu.VMEM((2, BPW, PADW), jnp.float32),
            pltpu.VMEM((2, HIDDEN, BPW), jnp.float32),
            pltpu.VMEM((SEQ, HIDDEN), jnp.float32),
            pltpu.VMEM((HIDDEN,), jnp.float32),
            pltpu.VMEM((HIDDEN,), jnp.float32),
            pltpu.SemaphoreType.DMA,
            pltpu.SemaphoreType.DMA,
            pltpu.SemaphoreType.DMA,
            pltpu.SemaphoreType.DMA,
        ],
        compiler_params=pltpu.CompilerParams(use_tc_tiling_on_sc=True,
                                             needs_layout_passes=False),
    )
    out = k(seqT, word128, pos_table, gamma, beta)
    return jnp.transpose(out, (2, 0, 1))


# speedup vs baseline: 1.9801x; 1.9801x over previous
import jax
import jax.numpy as jnp
from jax import lax
from jax.experimental import pallas as pl
from jax.experimental.pallas import tpu as pltpu
from jax.experimental.pallas import tpu_sc as plsc

HIDDEN = 64
SEQ = 200
BATCH = 4096
EPS = 1e-12

NC = 2
NS = 16
NW = NC * NS
BPW = BATCH // NW
L = 16
NV = HIDDEN // L
PADW = 128
YSTR = HIDDEN + 1

_GATHER_DNUMS = lax.GatherDimensionNumbers(
    offset_dims=(), collapsed_slice_dims=(0,), start_index_map=(0,))


def _lane_perm(x, perm):
    return lax.gather(x, perm[:, None], dimension_numbers=_GATHER_DNUMS,
                      slice_sizes=(1,),
                      mode=lax.GatherScatterMode.PROMISE_IN_BOUNDS)


def _allsum(x):
    lane = lax.iota(jnp.int32, L)
    for sh in (8, 4, 2, 1):
        x = x + _lane_perm(x, lane ^ sh)
    return x


def _rsqrt(v):
    i = lax.bitcast_convert_type(v, jnp.int32)
    y = lax.bitcast_convert_type(jnp.int32(0x5F3759DF) - (i >> 1),
                                 jnp.float32)
    for _ in range(2):
        y = y * (1.5 - 0.5 * v * y * y)
    return y


def _body(seqT_hbm, word_hbm, pos_hbm, gamma_hbm, beta_hbm, out_hbm,
          idx_v, rows_v, ybuf_v, slab_v, pos_v, gam_v, bet_v,
          sg0, sg1, so0, so1):
    wid = lax.axis_index("s") * NC + lax.axis_index("c")
    b0 = wid * BPW
    sems_g = (sg0, sg1)
    sems_o = (so0, so1)

    pltpu.sync_copy(pos_hbm, pos_v)
    pltpu.sync_copy(gamma_hbm, gam_v)
    pltpu.sync_copy(beta_hbm, bet_v)
    gam = [gam_v[pl.ds(L * j, L)] for j in range(NV)]
    bet = [bet_v[pl.ds(L * j, L)] for j in range(NV)]
    lane = lax.iota(jnp.int32, L)
    lane_y = lane * YSTR

    def stage_and_fire(s, pb):
        pltpu.sync_copy(seqT_hbm.at[s, pl.ds(b0, BPW)], idx_v.at[pb])
        pltpu.async_copy(word_hbm.at[idx_v.at[pb]], rows_v.at[pb],
                         sems_g[pb])

    def wait_gather(pb):
        pltpu.make_async_copy(word_hbm.at[idx_v.at[pb]], rows_v.at[pb],
                              sems_g[pb]).wait()

    def fire_out(s, pb):
        pltpu.async_copy(slab_v.at[pb],
                         out_hbm.at[s, slice(None), pl.ds(b0, BPW)],
                         sems_o[pb])

    def wait_out(pb):
        pltpu.make_async_copy(slab_v.at[pb],
                              out_hbm.at[0, slice(None), pl.ds(b0, BPW)],
                              sems_o[pb]).wait()

    def compute(s, pb):
        posv = [pos_v[s, pl.ds(L * j, L)] for j in range(NV)]

        @plsc.parallel_loop(0, BPW, 1, unroll=8)
        def _(t):
            x = [rows_v[pb, t, pl.ds(L * j, L)] + posv[j] for j in range(NV)]
            sm = (x[0] + x[1]) + (x[2] + x[3])
            q = (x[0] * x[0] + x[1] * x[1]) + (x[2] * x[2] + x[3] * x[3])
            mean = _allsum(sm) * (1.0 / HIDDEN)
            var = _allsum(q) * (1.0 / HIDDEN) - mean * mean
            rstd = _rsqrt(var + EPS)
            for j in range(NV):
                xn = (x[j] - mean) * rstd
                ybuf_v[pl.ds(t * YSTR + L * j, L)] = xn * gam[j] + bet[j]

        @plsc.parallel_loop(0, HIDDEN, 1, unroll=4)
        def _(h):
            for g in range(BPW // L):
                idx = lane_y + (g * (L * YSTR) + h)
                slab_v[pb, h, pl.ds(L * g, L)] = plsc.load_gather(
                    ybuf_v, [idx])

    stage_and_fire(0, 0)

    def pos_step(s, pb):
        @pl.when(s < SEQ - 1)
        def _():
            stage_and_fire(s + 1, 1 - pb)

        @pl.when(s >= 2)
        def _():
            wait_out(pb)
        wait_gather(pb)
        compute(s, pb)
        fire_out(s, pb)

    def iter_m(m, _):
        pos_step(2 * m, 0)
        pos_step(2 * m + 1, 1)
        return 0

    lax.fori_loop(0, SEQ // 2, iter_m, 0)
    wait_out(0)
    wait_out(1)


def kernel(seq, word_table, pos_table, gamma, beta):
    seqT = seq.astype(jnp.int32).T
    word128 = jnp.pad(word_table, ((0, 0), (0, PADW - HIDDEN)))
    mesh = plsc.VectorSubcoreMesh(core_axis_name="c", subcore_axis_name="s",
                                  num_cores=NC, num_subcores=NS)
    k = pl.kernel(
        _body,
        out_type=jax.ShapeDtypeStruct((SEQ, HIDDEN, BATCH), jnp.float32),
        mesh=mesh,
        scratch_types=[
            pltpu.VMEM((2, BPW), jnp.int32),
            pltpu.VMEM((2, BPW, PADW), jnp.float32),
            pltpu.VMEM((BPW * YSTR,), jnp.float32),
            pltpu.VMEM((2, HIDDEN, BPW), jnp.float32),
            pltpu.VMEM((SEQ, HIDDEN), jnp.float32),
            pltpu.VMEM((HIDDEN,), jnp.float32),
            pltpu.VMEM((HIDDEN,), jnp.float32),
            pltpu.SemaphoreType.DMA,
            pltpu.SemaphoreType.DMA,
            pltpu.SemaphoreType.DMA,
            pltpu.SemaphoreType.DMA,
        ],
        compiler_params=pltpu.CompilerParams(use_tc_tiling_on_sc=True,
                                             needs_layout_passes=False),
    )
    out = k(seqT, word128, pos_table, gamma, beta)
    return jnp.transpose(out, (2, 0, 1))
